# baseline (speedup 1.0000x reference)
import functools

import jax
import jax.numpy as jnp
from jax import lax
from jax.experimental import pallas as pl
from jax.experimental.pallas import tpu as pltpu
from jax.experimental.pallas import tpu_sc as plsc

B = 16384
ED = 64
FD = 128
HD = 128
NROWS = 1000000
NC = 2
NS = 16
NW = NC * NS
RPW = B // NW
L = 16
WAVE = 128
TBLK = 16384


TGRID = (NROWS + TBLK - 1) // TBLK
BSH = TBLK.bit_length() - 1
NBSC = 20
R0 = NBSC * TBLK
NPACK_SC = NBSC * (TBLK // 2)
NPACK_TC = (TGRID - NBSC) * (TBLK // 2)


def _tr_body(ut_ref, pt_ref, xu_ref, xp_ref):
    tu = ut_ref[...].T
    tp = pt_ref[...].T
    xu_ref[...] = jnp.concatenate([tu[:TBLK // 2], tu[TBLK // 2:]], axis=1)
    xp_ref[...] = jnp.concatenate([tp[:TBLK // 2], tp[TBLK // 2:]], axis=1)


def _transpose_tables_tc(utabT, ptabT):
    return pl.pallas_call(
        _tr_body,
        grid=(TGRID - NBSC,),
        in_specs=[
            pl.BlockSpec((ED, TBLK), lambda i: (0, i + NBSC)),
            pl.BlockSpec((ED, TBLK), lambda i: (0, i + NBSC)),
        ],
        out_specs=[
            pl.BlockSpec((TBLK // 2, 2 * ED), lambda i: (i, 0)),
            pl.BlockSpec((TBLK // 2, 2 * ED), lambda i: (i, 0)),
        ],
        out_shape=[
            jax.ShapeDtypeStruct((NPACK_TC, 2 * ED), jnp.float32),
            jax.ShapeDtypeStruct((NPACK_TC, 2 * ED), jnp.float32),
        ],
    )(utabT, ptabT)


@functools.cache
def _make_sc_transpose():
    mesh = plsc.VectorSubcoreMesh(core_axis_name="c", subcore_axis_name="s")
    pairs = NBSC * 64

    @functools.partial(
        pl.kernel,
        mesh=mesh,
        out_type=[
            jax.ShapeDtypeStruct((NPACK_SC, 2 * ED), jnp.float32),
            jax.ShapeDtypeStruct((NPACK_SC, 2 * ED), jnp.float32),
        ],
        scratch_types=[
            pltpu.VMEM((ED, 2 * ED), jnp.float32),
            pltpu.VMEM((ED, 2 * ED), jnp.float32),
            pltpu.VMEM((2 * ED, 2 * ED), jnp.float32),
            pltpu.SemaphoreType.DMA,
        ],
        compiler_params=pltpu.CompilerParams(needs_layout_passes=False),
    )
    def _sc_tr(utabT, ptabT, uout, pout, stage_a, stage_b, slab_v, sem):
        wid = lax.axis_index("s") * NC + lax.axis_index("c")
        iota = lax.broadcasted_iota(jnp.int32, (L,), 0)

        def one_table(tabT, out_hbm):
            def body(it, carry):
                p_id = it * NW + wid
                b = p_id // 64
                g = p_id % 64
                lane_a = pl.multiple_of((b * 128 + g) * 128, 128)
                lane_b = pl.multiple_of((b * 128 + g + 64) * 128, 128)
                pltpu.sync_copy(tabT.at[:, pl.ds(lane_a, 2 * ED)], stage_a)
                pltpu.sync_copy(tabT.at[:, pl.ds(lane_b, 2 * ED)], stage_b)

                def rowloop(l, c2):
                    lv = iota * 0 + l
                    for j in range(ED // L):
                        dv = iota + j * L
                        slab_v[l, pl.ds(j * L, L)] = plsc.load_gather(
                            stage_a, [dv, lv])
                        slab_v[l, pl.ds(ED + j * L, L)] = plsc.load_gather(
                            stage_b, [dv, lv])
                    return c2

                lax.fori_loop(0, 2 * ED, rowloop, 0)
                pltpu.sync_copy(
                    slab_v,
                    out_hbm.at[pl.ds(b * (TBLK // 2) + g * 128, 2 * ED)])
                return carry

            lax.fori_loop(0, pairs // NW, body, 0)

        one_table(utabT, uout)
        one_table(ptabT, pout)

    return _sc_tr


@functools.cache
def _make_sc_gather():
    mesh = plsc.VectorSubcoreMesh(core_axis_name="c", subcore_axis_name="s")

    @functools.partial(
        pl.kernel,
        mesh=mesh,
        out_type=[
            jax.ShapeDtypeStruct((NW, RPW, ED), jnp.float32),
            jax.ShapeDtypeStruct((NW, RPW, ED), jnp.float32),
        ],
        scratch_types=[
            pltpu.VMEM((RPW,), jnp.int32),
            pltpu.VMEM((RPW,), jnp.int32),
            pltpu.VMEM((WAVE, 2 * ED), jnp.float32),
            pltpu.VMEM((RPW, ED), jnp.float32),
            pltpu.SemaphoreType.DMA,
        ],
    )
    def _sc_gather(uidx_hbm, pidx_hbm, utab_sc, ptab_sc, utab_tc, ptab_tc,
                   uout_hbm, pout_hbm, uidx_v, pidx_v, pair_v, rows_v, sem):
        wid = lax.axis_index("s") * NC + lax.axis_index("c")
        pltpu.sync_copy(uidx_hbm.at[pl.ds(wid * RPW, RPW)], uidx_v)
        pltpu.sync_copy(pidx_hbm.at[pl.ds(wid * RPW, RPW)], pidx_v)

        def one_table(idx_v, tab_sc, tab_tc, out_hbm):
            for w in range(RPW // WAVE):
                def issue(g, carry):
                    vec = idx_v[pl.ds(w * WAVE + g * L, L)]
                    for lane in range(L):
                        r = vec[lane]
                        k = (r >> BSH) * (TBLK // 2) + (r & (TBLK // 2 - 1))
                        dst = pair_v.at[pl.ds(g * L + lane, 1)]

                        @pl.when(r < R0)
                        def _():
                            pltpu.async_copy(tab_sc.at[pl.ds(k, 1)], dst, sem)

                        @pl.when(r >= R0)
                        def _():
                            pltpu.async_copy(
                                tab_tc.at[pl.ds(k - NPACK_SC, 1)], dst, sem)
                    return carry

                lax.fori_loop(0, WAVE // L, issue, 0)

                def drain(i, carry):
                    pltpu.make_async_copy(tab_sc.at[pl.ds(0, 1)],
                                          pair_v.at[pl.ds(i, 1)], sem).wait()
                    return carry

                lax.fori_loop(0, WAVE, drain, 0)

                def extract(g, carry):
                    vec = idx_v[pl.ds(w * WAVE + g * L, L)]
                    for lane in range(L):
                        off = ((vec[lane] >> (BSH - 1)) & 1) * ED
                        i = g * L + lane
                        for j in range(ED // L):
                            rows_v[w * WAVE + i, pl.ds(j * L, L)] = (
                                pair_v[i, pl.ds(off + j * L, L)])
                    return carry

                lax.fori_loop(0, WAVE // L, extract, 0)
            pltpu.sync_copy(rows_v, out_hbm.at[wid])

        one_table(uidx_v, utab_sc, utab_tc, uout_hbm)
        one_table(pidx_v, ptab_sc, ptab_tc, pout_hbm)

    return _sc_gather


def _mlp_body(u_ref, p_ref, f_ref, w1a_ref, w1b_ref, w1c_ref, b1_ref,
              w2_ref, b2_ref, w3_ref, b3_ref, o_ref):
    dot = functools.partial(jnp.dot, preferred_element_type=jnp.float32,
                            precision=lax.Precision.DEFAULT)
    h = dot(u_ref[...], w1a_ref[...])
    h = h + dot(p_ref[...], w1b_ref[...])
    h = h + dot(f_ref[...], w1c_ref[...])
    h = jnp.maximum(h + b1_ref[...], 0.0)
    h = jnp.maximum(dot(h, w2_ref[...]) + b2_ref[...], 0.0)
    v = jnp.sum(h * w3_ref[...], axis=1) + b3_ref[0, 0]
    o_ref[0, 0, :] = 1.0 / (1.0 + jnp.exp(-v))


def _mlp(u, p, f, w1a, w1b, w1c, b1, w2, b2, w3r, b3s, bb):
    grid = B // bb
    full = lambda shape: pl.BlockSpec(shape, lambda i: (0, 0))
    return pl.pallas_call(
        _mlp_body,
        grid=(grid,),
        in_specs=[
            pl.BlockSpec((bb, ED), lambda i: (i, 0)),
            pl.BlockSpec((bb, ED), lambda i: (i, 0)),
            pl.BlockSpec((bb, FD), lambda i: (i, 0)),
            full((ED, HD)),
            full((ED, HD)),
            full((FD, HD)),
            full((1, HD)),
            full((HD, HD)),
            full((1, HD)),
            full((1, HD)),
            full((1, 1)),
        ],
        out_specs=pl.BlockSpec((1, 1, bb), lambda i: (i, 0, 0)),
        out_shape=jax.ShapeDtypeStruct((grid, 1, bb), jnp.float32),
    )(u, p, f, w1a, w1b, w1c, b1, w2, b2, w3r, b3s)


def kernel(user_indices, post_indices, features, user_table, post_table,
           W1, b1, W2, b2, W3, b3):
    uidx = user_indices.astype(jnp.int32)
    pidx = post_indices.astype(jnp.int32)
    utab_tc, ptab_tc = _transpose_tables_tc(user_table.T, post_table.T)
    utab_sc, ptab_sc = _make_sc_transpose()(user_table.T, post_table.T)
    uout, pout = _make_sc_gather()(uidx, pidx, utab_sc, ptab_sc,
                                   utab_tc, ptab_tc)
    u = uout.reshape(B, ED)
    p = pout.reshape(B, ED)
    out = _mlp(u, p, features,
               W1[:ED], W1[ED:2 * ED], W1[2 * ED:],
               b1.reshape(1, HD), W2, b2.reshape(1, HD),
               W3.reshape(1, HD), b3.reshape(1, 1), 2048)
    return out.reshape(B)


# speedup vs baseline: 2.6423x; 2.6423x over previous
import functools

import jax
import jax.numpy as jnp
from jax import lax
from jax.experimental import pallas as pl
from jax.experimental.pallas import tpu as pltpu
from jax.experimental.pallas import tpu_sc as plsc

B = 16384
ED = 64
FD = 128
HD = 128
NROWS = 1000000
NC = 2
NS = 16
NW = NC * NS
RPW = B // NW
L = 16
WAVE = 128
TBLK = 16384


TGRID = (NROWS + TBLK - 1) // TBLK
BSH = TBLK.bit_length() - 1
NBSC = 6
R0 = NBSC * TBLK
NPACK_SC = NBSC * (TBLK // 2)
NPACK_TC = (TGRID - NBSC) * (TBLK // 2)


def _tr_body(ut_ref, pt_ref, xu_ref, xp_ref):
    tu = ut_ref[...].T
    tp = pt_ref[...].T
    xu_ref[...] = jnp.concatenate([tu[:TBLK // 2], tu[TBLK // 2:]], axis=1)
    xp_ref[...] = jnp.concatenate([tp[:TBLK // 2], tp[TBLK // 2:]], axis=1)


def _transpose_tables_tc(utabT, ptabT):
    return pl.pallas_call(
        _tr_body,
        grid=(TGRID - NBSC,),
        in_specs=[
            pl.BlockSpec((ED, TBLK), lambda i: (0, i + NBSC)),
            pl.BlockSpec((ED, TBLK), lambda i: (0, i + NBSC)),
        ],
        out_specs=[
            pl.BlockSpec((TBLK // 2, 2 * ED), lambda i: (i, 0)),
            pl.BlockSpec((TBLK // 2, 2 * ED), lambda i: (i, 0)),
        ],
        out_shape=[
            jax.ShapeDtypeStruct((NPACK_TC, 2 * ED), jnp.float32),
            jax.ShapeDtypeStruct((NPACK_TC, 2 * ED), jnp.float32),
        ],
    )(utabT, ptabT)


@functools.cache
def _make_sc_transpose():
    mesh = plsc.VectorSubcoreMesh(core_axis_name="c", subcore_axis_name="s")
    pairs = NBSC * 64

    @functools.partial(
        pl.kernel,
        mesh=mesh,
        out_type=[
            jax.ShapeDtypeStruct((NPACK_SC, 2 * ED), jnp.float32),
            jax.ShapeDtypeStruct((NPACK_SC, 2 * ED), jnp.float32),
        ],
        scratch_types=[
            pltpu.VMEM((ED, 2 * ED), jnp.float32),
            pltpu.VMEM((ED, 2 * ED), jnp.float32),
            pltpu.VMEM((2 * ED, 2 * ED), jnp.float32),
            pltpu.SemaphoreType.DMA,
        ],
        compiler_params=pltpu.CompilerParams(needs_layout_passes=False),
    )
    def _sc_tr(utabT, ptabT, uout, pout, stage_a, stage_b, slab_v, sem):
        wid = lax.axis_index("s") * NC + lax.axis_index("c")
        iota = lax.broadcasted_iota(jnp.int32, (L,), 0)

        def one_table(tabT, out_hbm):
            def body(it, carry):
                p_id = it * NW + wid
                b = p_id // 64
                g = p_id % 64
                lane_a = pl.multiple_of((b * 128 + g) * 128, 128)
                lane_b = pl.multiple_of((b * 128 + g + 64) * 128, 128)
                pltpu.sync_copy(tabT.at[:, pl.ds(lane_a, 2 * ED)], stage_a)
                pltpu.sync_copy(tabT.at[:, pl.ds(lane_b, 2 * ED)], stage_b)

                def rowloop(l, c2):
                    lv = iota * 0 + l
                    for j in range(ED // L):
                        dv = iota + j * L
                        slab_v[l, pl.ds(j * L, L)] = plsc.load_gather(
                            stage_a, [dv, lv])
                        slab_v[l, pl.ds(ED + j * L, L)] = plsc.load_gather(
                            stage_b, [dv, lv])
                    return c2

                lax.fori_loop(0, 2 * ED, rowloop, 0)
                pltpu.sync_copy(
                    slab_v,
                    out_hbm.at[pl.ds(b * (TBLK // 2) + g * 128, 2 * ED)])
                return carry

            lax.fori_loop(0, pairs // NW, body, 0)

        one_table(utabT, uout)
        one_table(ptabT, pout)

    return _sc_tr


@functools.cache
def _make_sc_gather():
    mesh = plsc.VectorSubcoreMesh(core_axis_name="c", subcore_axis_name="s")

    @functools.partial(
        pl.kernel,
        mesh=mesh,
        out_type=[
            jax.ShapeDtypeStruct((NW, RPW, ED), jnp.float32),
            jax.ShapeDtypeStruct((NW, RPW, ED), jnp.float32),
        ],
        scratch_types=[
            pltpu.VMEM((RPW,), jnp.int32),
            pltpu.VMEM((RPW,), jnp.int32),
            pltpu.VMEM((WAVE, 2 * ED), jnp.float32),
            pltpu.VMEM((RPW, ED), jnp.float32),
            pltpu.SemaphoreType.DMA,
        ],
    )
    def _sc_gather(uidx_hbm, pidx_hbm, utab_sc, ptab_sc, utab_tc, ptab_tc,
                   uout_hbm, pout_hbm, uidx_v, pidx_v, pair_v, rows_v, sem):
        wid = lax.axis_index("s") * NC + lax.axis_index("c")
        pltpu.sync_copy(uidx_hbm.at[pl.ds(wid * RPW, RPW)], uidx_v)
        pltpu.sync_copy(pidx_hbm.at[pl.ds(wid * RPW, RPW)], pidx_v)

        def one_table(idx_v, tab_sc, tab_tc, out_hbm):
            for w in range(RPW // WAVE):
                def issue(g, carry):
                    vec = idx_v[pl.ds(w * WAVE + g * L, L)]
                    for lane in range(L):
                        r = vec[lane]
                        k = (r >> BSH) * (TBLK // 2) + (r & (TBLK // 2 - 1))
                        dst = pair_v.at[pl.ds(g * L + lane, 1)]

                        @pl.when(r < R0)
                        def _():
                            pltpu.async_copy(tab_sc.at[pl.ds(k, 1)], dst, sem)

                        @pl.when(r >= R0)
                        def _():
                            pltpu.async_copy(
                                tab_tc.at[pl.ds(k - NPACK_SC, 1)], dst, sem)
                    return carry

                lax.fori_loop(0, WAVE // L, issue, 0)

                def drain(i, carry):
                    pltpu.make_async_copy(tab_sc.at[pl.ds(0, 1)],
                                          pair_v.at[pl.ds(i, 1)], sem).wait()
                    return carry

                lax.fori_loop(0, WAVE, drain, 0)

                def extract(g, carry):
                    vec = idx_v[pl.ds(w * WAVE + g * L, L)]
                    for lane in range(L):
                        off = ((vec[lane] >> (BSH - 1)) & 1) * ED
                        i = g * L + lane
                        for j in range(ED // L):
                            rows_v[w * WAVE + i, pl.ds(j * L, L)] = (
                                pair_v[i, pl.ds(off + j * L, L)])
                    return carry

                lax.fori_loop(0, WAVE // L, extract, 0)
            pltpu.sync_copy(rows_v, out_hbm.at[wid])

        one_table(uidx_v, utab_sc, utab_tc, uout_hbm)
        one_table(pidx_v, ptab_sc, ptab_tc, pout_hbm)

    return _sc_gather


def _mlp_body(u_ref, p_ref, f_ref, w1a_ref, w1b_ref, w1c_ref, b1_ref,
              w2_ref, b2_ref, w3_ref, b3_ref, o_ref):
    dot = functools.partial(jnp.dot, preferred_element_type=jnp.float32,
                            precision=lax.Precision.DEFAULT)
    h = dot(u_ref[...], w1a_ref[...])
    h = h + dot(p_ref[...], w1b_ref[...])
    h = h + dot(f_ref[...], w1c_ref[...])
    h = jnp.maximum(h + b1_ref[...], 0.0)
    h = jnp.maximum(dot(h, w2_ref[...]) + b2_ref[...], 0.0)
    v = jnp.sum(h * w3_ref[...], axis=1) + b3_ref[0, 0]
    o_ref[0, 0, :] = 1.0 / (1.0 + jnp.exp(-v))


def _mlp(u, p, f, w1a, w1b, w1c, b1, w2, b2, w3r, b3s, bb):
    grid = B // bb
    full = lambda shape: pl.BlockSpec(shape, lambda i: (0, 0))
    return pl.pallas_call(
        _mlp_body,
        grid=(grid,),
        in_specs=[
            pl.BlockSpec((bb, ED), lambda i: (i, 0)),
            pl.BlockSpec((bb, ED), lambda i: (i, 0)),
            pl.BlockSpec((bb, FD), lambda i: (i, 0)),
            full((ED, HD)),
            full((ED, HD)),
            full((FD, HD)),
            full((1, HD)),
            full((HD, HD)),
            full((1, HD)),
            full((1, HD)),
            full((1, 1)),
        ],
        out_specs=pl.BlockSpec((1, 1, bb), lambda i: (i, 0, 0)),
        out_shape=jax.ShapeDtypeStruct((grid, 1, bb), jnp.float32),
    )(u, p, f, w1a, w1b, w1c, b1, w2, b2, w3r, b3s)


def kernel(user_indices, post_indices, features, user_table, post_table,
           W1, b1, W2, b2, W3, b3):
    uidx = user_indices.astype(jnp.int32)
    pidx = post_indices.astype(jnp.int32)
    utab_tc, ptab_tc = _transpose_tables_tc(user_table.T, post_table.T)
    utab_sc, ptab_sc = _make_sc_transpose()(user_table.T, post_table.T)
    uout, pout = _make_sc_gather()(uidx, pidx, utab_sc, ptab_sc,
                                   utab_tc, ptab_tc)
    u = uout.reshape(B, ED)
    p = pout.reshape(B, ED)
    out = _mlp(u, p, features,
               W1[:ED], W1[ED:2 * ED], W1[2 * ED:],
               b1.reshape(1, HD), W2, b2.reshape(1, HD),
               W3.reshape(1, HD), b3.reshape(1, 1), 2048)
    return out.reshape(B)


# speedup vs baseline: 2.6596x; 1.0065x over previous
import functools

import jax
import jax.numpy as jnp
from jax import lax
from jax.experimental import pallas as pl
from jax.experimental.pallas import tpu as pltpu
from jax.experimental.pallas import tpu_sc as plsc

B = 16384
ED = 64
FD = 128
HD = 128
NROWS = 1000000
NC = 2
NS = 16
NW = NC * NS
RPW = B // NW
L = 16
WAVE = 128
TBLK = 16384


TGRID = (NROWS + TBLK - 1) // TBLK
NPACK = TGRID * (TBLK // 2)
BSH = TBLK.bit_length() - 1


def _tr_body(ut_ref, pt_ref, xu_ref, xp_ref):
    tu = ut_ref[...].T
    tp = pt_ref[...].T
    xu_ref[...] = jnp.concatenate([tu[:TBLK // 2], tu[TBLK // 2:]], axis=1)
    xp_ref[...] = jnp.concatenate([tp[:TBLK // 2], tp[TBLK // 2:]], axis=1)


def _transpose_tables(utabT, ptabT):
    return pl.pallas_call(
        _tr_body,
        grid=(TGRID,),
        in_specs=[
            pl.BlockSpec((ED, TBLK), lambda i: (0, i)),
            pl.BlockSpec((ED, TBLK), lambda i: (0, i)),
        ],
        out_specs=[
            pl.BlockSpec((TBLK // 2, 2 * ED), lambda i: (i, 0)),
            pl.BlockSpec((TBLK // 2, 2 * ED), lambda i: (i, 0)),
        ],
        out_shape=[
            jax.ShapeDtypeStruct((NPACK, 2 * ED), jnp.float32),
            jax.ShapeDtypeStruct((NPACK, 2 * ED), jnp.float32),
        ],
    )(utabT, ptabT)


@functools.cache
def _make_sc_gather():
    mesh = plsc.VectorSubcoreMesh(core_axis_name="c", subcore_axis_name="s")

    @functools.partial(
        pl.kernel,
        mesh=mesh,
        out_type=[
            jax.ShapeDtypeStruct((NW, RPW, ED), jnp.float32),
            jax.ShapeDtypeStruct((NW, RPW, ED), jnp.float32),
        ],
        scratch_types=[
            pltpu.VMEM((RPW,), jnp.int32),
            pltpu.VMEM((RPW,), jnp.int32),
            pltpu.VMEM((WAVE, 2 * ED), jnp.float32),
            pltpu.VMEM((RPW, ED), jnp.float32),
            pltpu.SemaphoreType.DMA,
        ],
    )
    def _sc_gather(uidx_hbm, pidx_hbm, utab_hbm, ptab_hbm, uout_hbm, pout_hbm,
                   uidx_v, pidx_v, pair_v, rows_v, sem):
        wid = lax.axis_index("s") * NC + lax.axis_index("c")
        pltpu.sync_copy(uidx_hbm.at[pl.ds(wid * RPW, RPW)], uidx_v)
        pltpu.sync_copy(pidx_hbm.at[pl.ds(wid * RPW, RPW)], pidx_v)

        def one_table(idx_v, tab_hbm, out_hbm):
            for w in range(RPW // WAVE):
                def issue(g, carry):
                    vec = idx_v[pl.ds(w * WAVE + g * L, L)]
                    for lane in range(L):
                        r = vec[lane]
                        k = (r >> BSH) * (TBLK // 2) + (r & (TBLK // 2 - 1))
                        pltpu.async_copy(tab_hbm.at[pl.ds(k, 1)],
                                         pair_v.at[pl.ds(g * L + lane, 1)], sem)
                    return carry

                lax.fori_loop(0, WAVE // L, issue, 0)

                def drain(i, carry):
                    pltpu.make_async_copy(tab_hbm.at[pl.ds(0, 1)],
                                          pair_v.at[pl.ds(i, 1)], sem).wait()
                    return carry

                lax.fori_loop(0, WAVE, drain, 0)

                def extract(g, carry):
                    vec = idx_v[pl.ds(w * WAVE + g * L, L)]
                    for lane in range(L):
                        off = ((vec[lane] >> (BSH - 1)) & 1) * ED
                        i = g * L + lane
                        for j in range(ED // L):
                            rows_v[w * WAVE + i, pl.ds(j * L, L)] = (
                                pair_v[i, pl.ds(off + j * L, L)])
                    return carry

                lax.fori_loop(0, WAVE // L, extract, 0)
            pltpu.sync_copy(rows_v, out_hbm.at[wid])

        one_table(uidx_v, utab_hbm, uout_hbm)
        one_table(pidx_v, ptab_hbm, pout_hbm)

    return _sc_gather


def _mlp_body(u_ref, p_ref, f_ref, w1a_ref, w1b_ref, w1c_ref, b1_ref,
              w2_ref, b2_ref, w3_ref, b3_ref, o_ref):
    dot = functools.partial(jnp.dot, preferred_element_type=jnp.float32,
                            precision=lax.Precision.DEFAULT)
    h = dot(u_ref[...], w1a_ref[...])
    h = h + dot(p_ref[...], w1b_ref[...])
    h = h + dot(f_ref[...], w1c_ref[...])
    h = jnp.maximum(h + b1_ref[...], 0.0)
    h = jnp.maximum(dot(h, w2_ref[...]) + b2_ref[...], 0.0)
    v = jnp.sum(h * w3_ref[...], axis=1) + b3_ref[0, 0]
    o_ref[0, 0, :] = 1.0 / (1.0 + jnp.exp(-v))


def _mlp(u, p, f, w1a, w1b, w1c, b1, w2, b2, w3r, b3s, bb):
    grid = B // bb
    full = lambda shape: pl.BlockSpec(shape, lambda i: (0, 0))
    return pl.pallas_call(
        _mlp_body,
        grid=(grid,),
        in_specs=[
            pl.BlockSpec((bb, ED), lambda i: (i, 0)),
            pl.BlockSpec((bb, ED), lambda i: (i, 0)),
            pl.BlockSpec((bb, FD), lambda i: (i, 0)),
            full((ED, HD)),
            full((ED, HD)),
            full((FD, HD)),
            full((1, HD)),
            full((HD, HD)),
            full((1, HD)),
            full((1, HD)),
            full((1, 1)),
        ],
        out_specs=pl.BlockSpec((1, 1, bb), lambda i: (i, 0, 0)),
        out_shape=jax.ShapeDtypeStruct((grid, 1, bb), jnp.float32),
    )(u, p, f, w1a, w1b, w1c, b1, w2, b2, w3r, b3s)


def kernel(user_indices, post_indices, features, user_table, post_table,
           W1, b1, W2, b2, W3, b3):
    uidx = user_indices.astype(jnp.int32)
    pidx = post_indices.astype(jnp.int32)
    utab_rm, ptab_rm = _transpose_tables(user_table.T, post_table.T)
    uout, pout = _make_sc_gather()(uidx, pidx, utab_rm, ptab_rm)
    u = uout.reshape(B, ED)
    p = pout.reshape(B, ED)
    out = _mlp(u, p, features,
               W1[:ED], W1[ED:2 * ED], W1[2 * ED:],
               b1.reshape(1, HD), W2, b2.reshape(1, HD),
               W3.reshape(1, HD), b3.reshape(1, 1), 2048)
    return out.reshape(B)
